# baseline (speedup 1.0000x reference)
import functools

import jax
import jax.numpy as jnp
from jax import lax
from jax.experimental import pallas as pl
from jax.experimental.pallas import tpu as pltpu
from jax.experimental.pallas import tpu_sc as plsc

_N_USERS = 500000
_B = 16384
_D = 64

_info = plsc.get_sparse_core_info()
_NC, _NS, _L = _info.num_cores, _info.num_subcores, _info.num_lanes
_NW = _NC * _NS
_BPW = _B // _NW

_mesh = plsc.VectorSubcoreMesh(core_axis_name="c", subcore_axis_name="s")


@functools.partial(
    pl.kernel,
    mesh=_mesh,
    out_type=[jax.ShapeDtypeStruct((_B, _D), jnp.float32) for _ in range(3)],
    scratch_types=[
        pltpu.VMEM((_BPW,), jnp.int32),
        pltpu.VMEM((_BPW,), jnp.int32),
        pltpu.VMEM((_BPW,), jnp.int32),
        pltpu.VMEM((_BPW, _D), jnp.float32),
        pltpu.VMEM((_BPW, _D), jnp.float32),
        pltpu.VMEM((_BPW, _D), jnp.float32),
        pltpu.SemaphoreType.DMA,
    ],
    compiler_params=pltpu.CompilerParams(use_tc_tiling_on_sc=False),
)
def _gather3(users_hbm, pos_hbm, neg_hbm, table_hbm,
             u_out, p_out, n_out,
             iu_v, ip_v, in_v, ru_v, rp_v, rn_v, sem):
    wid = lax.axis_index("s") * _NC + lax.axis_index("c")
    base = wid * _BPW
    sl = pl.ds(base, _BPW)

    pltpu.sync_copy(users_hbm.at[sl], iu_v)
    pltpu.sync_copy(pos_hbm.at[sl], ip_v)
    pltpu.sync_copy(neg_hbm.at[sl], in_v)

    for i in range(_BPW // _L):
        s = pl.ds(i * _L, _L)
        ip_v[s] = ip_v[s] + _N_USERS
        in_v[s] = in_v[s] + _N_USERS

    cu = pltpu.async_copy(table_hbm.at[iu_v], ru_v, sem)
    cp = pltpu.async_copy(table_hbm.at[ip_v], rp_v, sem)
    cn = pltpu.async_copy(table_hbm.at[in_v], rn_v, sem)
    cu.wait()
    cp.wait()
    cn.wait()

    pltpu.sync_copy(ru_v, u_out.at[sl])
    pltpu.sync_copy(rp_v, p_out.at[sl])
    pltpu.sync_copy(rn_v, n_out.at[sl])


_RB = 2048


def _l2_body(u_ref, p_ref, n_ref, o_ref):
    u = u_ref[...]
    p = p_ref[...]
    n = n_ref[...]
    o_ref[...] = jnp.sum(u * u + p * p + n * n, axis=1, keepdims=True)


_l2_call = pl.pallas_call(
    _l2_body,
    grid=(_B // _RB,),
    in_specs=[pl.BlockSpec((_RB, _D), lambda i: (i, 0)) for _ in range(3)],
    out_specs=pl.BlockSpec((_RB, 1), lambda i: (i, 0)),
    out_shape=jax.ShapeDtypeStruct((_B, 1), jnp.float32),
)


def kernel(users, pos_items, neg_items, embedding_weight):
    users_e, pos_e, neg_e = _gather3(users, pos_items, neg_items,
                                     embedding_weight)
    l2 = _l2_call(users_e, pos_e, neg_e).reshape(_B)
    return users_e, pos_e, neg_e, l2


# speedup vs baseline: 1.6479x; 1.6479x over previous
import functools

import jax
import jax.numpy as jnp
from jax import lax
from jax.experimental import pallas as pl
from jax.experimental.pallas import tpu as pltpu
from jax.experimental.pallas import tpu_sc as plsc

_N_USERS = 500000
_B = 16384
_D = 64

_info = plsc.get_sparse_core_info()
_NC, _NS, _L = _info.num_cores, _info.num_subcores, _info.num_lanes
_NW = _NC * _NS
_BPW = _B // _NW

_mesh = plsc.VectorSubcoreMesh(core_axis_name="c", subcore_axis_name="s")


@functools.partial(
    pl.kernel,
    mesh=_mesh,
    out_type=[jax.ShapeDtypeStruct((_B, _D), jnp.float32) for _ in range(3)],
    scratch_types=[
        pltpu.VMEM((_BPW,), jnp.int32),
        pltpu.VMEM((_BPW, _D), jnp.float32),
        pltpu.SemaphoreType.DMA,
    ],
    compiler_params=pltpu.CompilerParams(use_tc_tiling_on_sc=True,
                                         needs_layout_passes=False),
)
def _gather3(users_hbm, pos_hbm, neg_hbm, table_hbm,
             u_out, p_out, n_out,
             idx_v, rows_v, sem):
    wid = lax.axis_index("s") * _NC + lax.axis_index("c")
    base = wid * _BPW
    sl = pl.ds(base, _BPW)

    for idx_hbm, out_hbm, off in ((users_hbm, u_out, 0),
                                  (pos_hbm, p_out, _N_USERS),
                                  (neg_hbm, n_out, _N_USERS)):
        pltpu.sync_copy(idx_hbm.at[sl], idx_v)

        lane = lax.iota(jnp.int32, _L)

        def fire(g, _, off=off):
            iv = idx_v[pl.ds(g * _L, _L)] + off
            for k in range(_L):
                row = jnp.sum(jnp.where(lane == k, iv, 0))
                pltpu.async_copy(table_hbm.at[pl.ds(row, 1)],
                                 rows_v.at[pl.ds(g * _L + k, 1)], sem)
            return 0

        lax.fori_loop(0, _BPW // _L, fire, 0)
        pltpu.make_async_copy(table_hbm.at[pl.ds(0, _BPW)], rows_v, sem).wait()
        pltpu.sync_copy(rows_v, out_hbm.at[sl])


_RB = 2048


def _l2_body(u_ref, p_ref, n_ref, o_ref):
    u = u_ref[...]
    p = p_ref[...]
    n = n_ref[...]
    o_ref[...] = jnp.sum(u * u + p * p + n * n, axis=1, keepdims=True)


_l2_call = pl.pallas_call(
    _l2_body,
    grid=(_B // _RB,),
    in_specs=[pl.BlockSpec((_RB, _D), lambda i: (i, 0)) for _ in range(3)],
    out_specs=pl.BlockSpec((_RB, 1), lambda i: (i, 0)),
    out_shape=jax.ShapeDtypeStruct((_B, 1), jnp.float32),
)


def kernel(users, pos_items, neg_items, embedding_weight):
    users_e, pos_e, neg_e = _gather3(users, pos_items, neg_items,
                                     embedding_weight)
    l2 = _l2_call(users_e, pos_e, neg_e).reshape(_B)
    return users_e, pos_e, neg_e, l2
